# baseline (speedup 1.0000x reference)
import functools

import jax
import jax.numpy as jnp
from jax import lax
from jax.experimental import pallas as pl
from jax.experimental.pallas import tpu as pltpu
from jax.experimental.pallas import tpu_sc as plsc

_B = 4096
_D = 64
_C = 5024

_BM = 256
_NB = _B // _BM


@functools.lru_cache(maxsize=1)
def _make_gather3():
    info = plsc.get_sparse_core_info()
    nc, ns = info.num_cores, info.num_subcores
    nw = nc * ns
    bpw = _B // nw
    mesh = plsc.VectorSubcoreMesh(core_axis_name="c", subcore_axis_name="s")

    @functools.partial(
        pl.kernel,
        mesh=mesh,
        compiler_params=pltpu.CompilerParams(use_tc_tiling_on_sc=False),
        out_type=[jax.ShapeDtypeStruct((_B, _D), jnp.float32)] * 3,
        scratch_types=[
            pltpu.VMEM((bpw,), jnp.int32),
            pltpu.VMEM((bpw,), jnp.int32),
            pltpu.VMEM((bpw, _D), jnp.float32),
            pltpu.VMEM((bpw, _D), jnp.float32),
            pltpu.VMEM((bpw, _D), jnp.float32),
            pltpu.SemaphoreType.DMA,
            pltpu.SemaphoreType.DMA,
            pltpu.SemaphoreType.DMA,
        ],
    )
    def gather3(uid_hbm, iid_hbm, ut_hbm, itc_hbm, itn_hbm,
                u_out, i_out, n_out,
                uidx, iidx, urows, irows, nrows, su, si, sn):
        wid = lax.axis_index("s") * nc + lax.axis_index("c")
        base = wid * bpw
        pltpu.sync_copy(uid_hbm.at[pl.ds(base, bpw)], uidx)
        pltpu.sync_copy(iid_hbm.at[pl.ds(base, bpw)], iidx)
        cu = pltpu.async_copy(ut_hbm.at[uidx], urows, su)
        ci = pltpu.async_copy(itc_hbm.at[iidx], irows, si)
        cn = pltpu.async_copy(itn_hbm.at[iidx], nrows, sn)
        cu.wait()
        ci.wait()
        cn.wait()
        pltpu.sync_copy(urows, u_out.at[pl.ds(base, bpw)])
        pltpu.sync_copy(irows, i_out.at[pl.ds(base, bpw)])
        pltpu.sync_copy(nrows, n_out.at[pl.ds(base, bpw)])

    return gather3


def _combine_body(content_ref, u_ref, i_ref, n_ref, wc_ref, cb_ref, cw_ref,
                  hybw_ref, cbias_ref, hbias_ref, out_ref):
    mat = lax.dot_general(
        content_ref[...], wc_ref[...],
        (((1,), (1,)), ((), ())),
        preferred_element_type=jnp.float32)
    hw0 = hybw_ref[0, 0]
    hw1 = hybw_ref[0, 1]
    content_pred = jnp.sum(n_ref[...] * (mat + cb_ref[...]), axis=1,
                           keepdims=True)
    collab_pred = jnp.sum(u_ref[...] * i_ref[...] * cw_ref[...], axis=1,
                          keepdims=True) + cbias_ref[0, 0]
    out_ref[...] = hw0 * collab_pred + hw1 * content_pred + hbias_ref[0, 0]


def _combine(content_features, u_emb, i_emb, n_emb, content_fc_w,
             content_fc_b, collab_fc_w, hybrid_fc_w, collab_fc_b,
             hybrid_fc_b):
    emb_spec = pl.BlockSpec((_BM, _D), lambda i: (i, 0))
    full = lambda shape: pl.BlockSpec(shape, lambda i: (0, 0))
    return pl.pallas_call(
        _combine_body,
        grid=(_NB,),
        in_specs=[
            pl.BlockSpec((_BM, _C), lambda i: (i, 0)),
            emb_spec, emb_spec, emb_spec,
            full((_D, _C)),
            full((1, _D)),
            full((1, _D)),
            full((1, 2)),
            full((1, 1)),
            full((1, 1)),
        ],
        out_specs=pl.BlockSpec((_BM, 1), lambda i: (i, 0)),
        out_shape=jax.ShapeDtypeStruct((_B, 1), jnp.float32),
        compiler_params=pltpu.CompilerParams(
            dimension_semantics=("arbitrary",)),
    )(content_features, u_emb, i_emb, n_emb, content_fc_w,
      content_fc_b.reshape(1, _D), collab_fc_w, hybrid_fc_w,
      collab_fc_b.reshape(1, 1), hybrid_fc_b.reshape(1, 1))


def kernel(user_id, item_id, content_features, user_table, item_table_collab,
           collab_fc_w, collab_fc_b, item_table_content, content_fc_w,
           content_fc_b, hybrid_fc_w, hybrid_fc_b):
    uid = user_id.astype(jnp.int32)
    iid = item_id.astype(jnp.int32)
    u_emb, i_emb, n_emb = _make_gather3()(
        uid, iid, user_table, item_table_collab, item_table_content)
    out = _combine(content_features, u_emb, i_emb, n_emb, content_fc_w,
                   content_fc_b, collab_fc_w, hybrid_fc_w, collab_fc_b,
                   hybrid_fc_b)
    return out.reshape(_B)
